# baseline (speedup 1.0000x reference)
import functools

import jax
import jax.numpy as jnp
from jax import lax
from jax.experimental import pallas as pl
from jax.experimental.pallas import tpu as pltpu
from jax.experimental.pallas import tpu_sc as plsc

N_NODES = 207
N_EDGES = 1722
L = 16
NB = 112
NEP = NB * L
BW = 256
NF = 12
XW = N_NODES * NF
NW = (XW + L - 1) // L
XV = NW * L
IW = 2 * XW


def _vtanh(a):
    e = jnp.exp(jnp.abs(a) * 2.0)
    return jnp.sign(a) * (1.0 - 2.0 / (e + 1.0))


def _body(x_hbm, ind_hbm, ei_hbm, ej_hbm, nt_hbm, wr_hbm, wd_hbm, wra_hbm,
          wda_hbm, br_hbm, bd_hbm, bra_hbm, bda_hbm, out_hbm,
          idxw, ei_v, ej_v, nt_v, wr_v, wd_v, wra_v, wda_v,
          br_v, bd_v, bra_v, bda_v,
          x_v2, accR, accRa, accDD, cR, cRa, cDD, out_v,
          sem, sem3):
    cidx = lax.axis_index("c")
    sidx = lax.axis_index("s")
    wid = sidx * 2 + cidx
    b0 = wid * 2

    pltpu.sync_copy(ind_hbm.at[pl.ds(wid * 8, 8)], idxw)

    idx2 = idxw.at[pl.ds(0, 2)]
    copies = [
        pltpu.async_copy(wr_hbm.at[idx2], wr_v, sem),
        pltpu.async_copy(wd_hbm.at[idx2], wd_v, sem),
        pltpu.async_copy(wra_hbm.at[idx2], wra_v, sem),
        pltpu.async_copy(wda_hbm.at[idx2], wda_v, sem),
        pltpu.async_copy(ei_hbm, ei_v, sem),
        pltpu.async_copy(ej_hbm, ej_v, sem),
        pltpu.async_copy(x_hbm.at[pl.ds(b0 * XW, 2 * XW)], x_v2, sem),
    ]
    copies2 = [
        pltpu.async_copy(br_hbm.at[idx2], br_v, sem3),
        pltpu.async_copy(bd_hbm.at[idx2], bd_v, sem3),
        pltpu.async_copy(bra_hbm.at[idx2], bra_v, sem3),
        pltpu.async_copy(bda_hbm.at[idx2], bda_v, sem3),
        pltpu.async_copy(nt_hbm, nt_v, sem3),
    ]

    zf = jnp.zeros((L,), jnp.float32)

    def _zero():
        @plsc.parallel_loop(0, NW, unroll=8)
        def _zero_acc(i):
            accR[pl.ds(i * L, L)] = zf
            accRa[pl.ds(i * L, L)] = zf
            accDD[pl.ds(i * L, L)] = zf

        @plsc.parallel_loop(0, 13, unroll=4)
        def _zero_c(i):
            cR[pl.ds(i * L, L)] = zf
            cRa[pl.ds(i * L, L)] = zf
            cDD[pl.ds(i * L, L)] = zf

    with jax.named_scope("ph_zero0"):
        _zero()
    with jax.named_scope("ph_wait0"):
        for c in copies:
            c.wait()

    for k in (0, 1):
        b = b0 + k
        ob = k * XW

        sc_e = jax.named_scope(f"ph_edges{k}"); sc_e.__enter__()
        @plsc.parallel_loop(0, NB, unroll=2)
        def _edges(bi):
            base = bi * L
            i16 = ei_v[pl.ds(base, L)]
            j16 = ej_v[pl.ds(base, L)]
            wr16 = wr_v[k, pl.ds(base, L)]
            wd16 = wd_v[k, pl.ds(base, L)]
            wra16 = wra_v[k, pl.ds(base, L)]
            wda16 = wda_v[k, pl.ds(base, L)]
            plsc.addupdate_scatter(cR, [j16], wr16)
            plsc.addupdate_scatter(cRa, [j16], wra16)
            plsc.addupdate_scatter(cDD, [i16], wd16 + wda16)
            wdd16 = wda16 - wd16
            ib = i16 * NF
            jb = j16 * NF
            for f in range(NF):
                ibf = ib + f
                jbf = jb + f
                xj = plsc.load_gather(x_v2, [jb + (ob + f)])
                xi = plsc.load_gather(x_v2, [ib + (ob + f)])
                plsc.addupdate_scatter(accR, [ibf], wr16 * xj)
                plsc.addupdate_scatter(accRa, [ibf], wra16 * xj)
                plsc.addupdate_scatter(accDD, [jbf], wdd16 * xi)

        sc_e.__exit__(None, None, None)
        if k == 0:
            for c in copies2:
                c.wait()
            for k2 in (0, 1):
                @plsc.parallel_loop(0, 13, unroll=4)
                def _bsum(i):
                    bd_v[k2, pl.ds(i * L, L)] = (bd_v[k2, pl.ds(i * L, L)]
                                                 + bda_v[k2, pl.ds(i * L, L)])

        ga = jnp.where(jnp.full((L,), b, jnp.int32) == 0, 0.0, 1.0)
        ks = jnp.full((L,), k, jnp.int32)

        sc_n = jax.named_scope(f"ph_nodes{k}"); sc_n.__enter__()
        @plsc.parallel_loop(0, NW, unroll=2)
        def _nodes(t):
            tb = pl.ds(t * L, L)
            nidx = nt_v[tb]
            xw = plsc.load_gather(x_v2, [jnp.minimum(t * L + lax.iota(jnp.int32, L),
                                                     XW - 1) + ob])
            r = plsc.load_gather(cR, [nidx]) * xw - accR[tb] \
                + plsc.load_gather(br_v, [ks, nidx])
            ra = plsc.load_gather(cRa, [nidx]) * ga * xw + accRa[tb] \
                + plsc.load_gather(bra_v, [ks, nidx])
            dd = plsc.load_gather(cDD, [nidx]) * xw + accDD[tb] \
                + plsc.load_gather(bd_v, [ks, nidx])
            out_v[pl.ds(ob + t * L, L)] = _vtanh(r) + _vtanh(ra) + dd + xw

        sc_n.__exit__(None, None, None)
        if k == 0:
            with jax.named_scope("ph_zero1"):
                _zero()

    with jax.named_scope("ph_out"):
            pltpu.sync_copy(out_v.at[pl.ds(0, 2 * XW)],
                        out_hbm.at[pl.ds(b0 * XW, 2 * XW)])


@jax.jit
def _run(xp, ind, ei, ej, nt, wr, wd, wra, wda, br, bd, bra, bda):
    f32 = jnp.float32
    i32 = jnp.int32
    fn = functools.partial(
        pl.kernel,
        out_type=jax.ShapeDtypeStruct((64 * XW,), f32),
        mesh=plsc.VectorSubcoreMesh(core_axis_name="c", subcore_axis_name="s"),
        compiler_params=pltpu.CompilerParams(needs_layout_passes=False),
        scratch_types=[
            pltpu.VMEM((8,), i32),
            pltpu.VMEM((NEP,), i32),
            pltpu.VMEM((NEP,), i32),
            pltpu.VMEM((XV,), i32),
            pltpu.VMEM((2, NEP), f32),
            pltpu.VMEM((2, NEP), f32),
            pltpu.VMEM((2, NEP), f32),
            pltpu.VMEM((2, NEP), f32),
            pltpu.VMEM((2, BW), f32),
            pltpu.VMEM((2, BW), f32),
            pltpu.VMEM((2, BW), f32),
            pltpu.VMEM((2, BW), f32),
            pltpu.VMEM((2 * XW,), f32),
            pltpu.VMEM((XV,), f32),
            pltpu.VMEM((XV,), f32),
            pltpu.VMEM((XV,), f32),
            pltpu.VMEM((208,), f32),
            pltpu.VMEM((208,), f32),
            pltpu.VMEM((208,), f32),
            pltpu.VMEM((XW + XV,), f32),
            pltpu.SemaphoreType.DMA,
            pltpu.SemaphoreType.DMA,
        ],
    )(_body)
    return fn(xp, ind, ei, ej, nt, wr, wd, wra, wda, br, bd, bra, bda)


def kernel(inputs, ind, edge_index, weight_react, weight_diff, weight_react_a,
           weight_diff_a, bias_reaction, bias_diffusion, bias_reaction_a,
           bias_diffusion_a):
    xp = inputs[:, 0, :, :].reshape(64 * XW)
    ep = jnp.pad(edge_index, ((0, 0), (0, NEP - N_EDGES)))
    nt = jnp.arange(NW * L, dtype=jnp.int32) // NF
    padw = lambda w: jnp.pad(w, ((0, 0), (0, NEP - N_EDGES)))
    padb = lambda b: jnp.pad(b, ((0, 0), (0, BW - N_NODES)))
    ind8 = jnp.pad(ind.astype(jnp.int32).reshape(32, 2), ((0, 0), (0, 6))).reshape(256)
    res = _run(xp, ind8, ep[0], ep[1], nt,
               padw(weight_react), padw(weight_diff),
               padw(weight_react_a), padw(weight_diff_a),
               padb(bias_reaction), padb(bias_diffusion),
               padb(bias_reaction_a), padb(bias_diffusion_a))
    return res.reshape(64, N_NODES, NF)


# speedup vs baseline: 1.1022x; 1.1022x over previous
import functools

import jax
import jax.numpy as jnp
from jax import lax
from jax.experimental import pallas as pl
from jax.experimental.pallas import tpu as pltpu
from jax.experimental.pallas import tpu_sc as plsc

N_NODES = 207
N_EDGES = 1722
L = 16
NB = 112
NEP = NB * L
BW = 256
NF = 12
XW = N_NODES * NF
NW = (XW + L - 1) // L
XV = NW * L
ST = NF + 1
X13 = 2704


def _vtanh(a):
    e = jnp.exp(jnp.abs(a) * 2.0)
    return jnp.sign(a) * (1.0 - 2.0 / (e + 1.0))


def _body(x_hbm, ind_hbm, ei_hbm, ej_hbm, nt_hbm, wr_hbm, wd_hbm, wra_hbm,
          wda_hbm, br_hbm, bd_hbm, bra_hbm, bda_hbm, out_hbm,
          idxw, ei_v, ej_v, nt_v, wr_v, wd_v, wra_v, wda_v,
          br_v, bd_v, bra_v, bda_v,
          x_v2, x13, accR, accRa, accDD, cR, cRa, cDD, out_v,
          sem, sem3):
    cidx = lax.axis_index("c")
    sidx = lax.axis_index("s")
    wid = sidx * 2 + cidx
    b0 = wid * 2

    pltpu.sync_copy(ind_hbm.at[pl.ds(wid * 8, 8)], idxw)

    idx2 = idxw.at[pl.ds(0, 2)]
    copies = [
        pltpu.async_copy(wr_hbm.at[idx2], wr_v, sem),
        pltpu.async_copy(wd_hbm.at[idx2], wd_v, sem),
        pltpu.async_copy(wra_hbm.at[idx2], wra_v, sem),
        pltpu.async_copy(wda_hbm.at[idx2], wda_v, sem),
        pltpu.async_copy(ei_hbm, ei_v, sem),
        pltpu.async_copy(ej_hbm, ej_v, sem),
        pltpu.async_copy(nt_hbm, nt_v, sem),
        pltpu.async_copy(x_hbm.at[pl.ds(b0 * XW, 2 * XW)], x_v2.at[pl.ds(0, 2 * XW)], sem),
    ]
    copies2 = [
        pltpu.async_copy(br_hbm.at[idx2], br_v, sem3),
        pltpu.async_copy(bd_hbm.at[idx2], bd_v, sem3),
        pltpu.async_copy(bra_hbm.at[idx2], bra_v, sem3),
        pltpu.async_copy(bda_hbm.at[idx2], bda_v, sem3),
    ]

    zf = jnp.zeros((L,), jnp.float32)

    def _zero():
        @plsc.parallel_loop(0, X13 // L, unroll=8)
        def _zero_acc(i):
            accR[pl.ds(i * L, L)] = zf
            accRa[pl.ds(i * L, L)] = zf
            accDD[pl.ds(i * L, L)] = zf

        @plsc.parallel_loop(0, 13, unroll=4)
        def _zero_c(i):
            cR[pl.ds(i * L, L)] = zf
            cRa[pl.ds(i * L, L)] = zf
            cDD[pl.ds(i * L, L)] = zf

    with jax.named_scope("ph_zero0"):
        _zero()
    with jax.named_scope("ph_wait0"):
        for c in copies:
            c.wait()

    iota16 = lax.iota(jnp.int32, L)
    for k in (0, 1):
        b = b0 + k
        ob = k * XW

        @plsc.parallel_loop(0, NW, unroll=4)
        def _relay(t):
            wv = t * L + iota16
            nidx = nt_v[pl.ds(t * L, L)]
            xval = x_v2[pl.ds(ob + t * L, L)]
            plsc.store_scatter(x13, [wv + nidx], xval)

        sc_e = jax.named_scope(f"ph_edges{k}"); sc_e.__enter__()
        @plsc.parallel_loop(0, NB, unroll=2)
        def _edges(bi):
            base = bi * L
            i16 = ei_v[pl.ds(base, L)]
            j16 = ej_v[pl.ds(base, L)]
            wr16 = wr_v[k, pl.ds(base, L)]
            wd16 = wd_v[k, pl.ds(base, L)]
            wra16 = wra_v[k, pl.ds(base, L)]
            wda16 = wda_v[k, pl.ds(base, L)]
            plsc.addupdate_scatter(cR, [j16], wr16)
            plsc.addupdate_scatter(cRa, [j16], wra16)
            plsc.addupdate_scatter(cDD, [i16], wd16 + wda16)
            wdd16 = wda16 - wd16
            ib = i16 * ST
            jb = j16 * ST
            for f in range(NF):
                ibf = ib + f
                jbf = jb + f
                xj = plsc.load_gather(x13, [jbf])
                xi = plsc.load_gather(x13, [ibf])
                plsc.addupdate_scatter(accR, [ibf], wr16 * xj)
                plsc.addupdate_scatter(accRa, [ibf], wra16 * xj)
                plsc.addupdate_scatter(accDD, [jbf], wdd16 * xi)

        sc_e.__exit__(None, None, None)
        if k == 0:
            for c in copies2:
                c.wait()
            for k2 in (0, 1):
                @plsc.parallel_loop(0, 13, unroll=4)
                def _bsum(i):
                    bd_v[k2, pl.ds(i * L, L)] = (bd_v[k2, pl.ds(i * L, L)]
                                                 + bda_v[k2, pl.ds(i * L, L)])

        ga = jnp.where(jnp.full((L,), b, jnp.int32) == 0, 0.0, 1.0)
        ks = jnp.full((L,), k, jnp.int32)

        sc_n = jax.named_scope(f"ph_nodes{k}"); sc_n.__enter__()
        @plsc.parallel_loop(0, NW, unroll=2)
        def _nodes(t):
            tb = pl.ds(t * L, L)
            nidx = nt_v[tb]
            aidx = t * L + iota16 + nidx
            xw = plsc.load_gather(x13, [aidx])
            r = plsc.load_gather(cR, [nidx]) * xw - plsc.load_gather(accR, [aidx]) \
                + plsc.load_gather(br_v, [ks, nidx])
            ra = plsc.load_gather(cRa, [nidx]) * ga * xw + plsc.load_gather(accRa, [aidx]) \
                + plsc.load_gather(bra_v, [ks, nidx])
            dd = plsc.load_gather(cDD, [nidx]) * xw + plsc.load_gather(accDD, [aidx]) \
                + plsc.load_gather(bd_v, [ks, nidx])
            out_v[pl.ds(ob + t * L, L)] = _vtanh(r) + _vtanh(ra) + dd + xw

        sc_n.__exit__(None, None, None)
        if k == 0:
            with jax.named_scope("ph_zero1"):
                _zero()

    with jax.named_scope("ph_out"):
            pltpu.sync_copy(out_v.at[pl.ds(0, 2 * XW)],
                        out_hbm.at[pl.ds(b0 * XW, 2 * XW)])


@jax.jit
def _run(xp, ind, ei, ej, nt, wr, wd, wra, wda, br, bd, bra, bda):
    f32 = jnp.float32
    i32 = jnp.int32
    fn = functools.partial(
        pl.kernel,
        out_type=jax.ShapeDtypeStruct((64 * XW,), f32),
        mesh=plsc.VectorSubcoreMesh(core_axis_name="c", subcore_axis_name="s"),
        compiler_params=pltpu.CompilerParams(needs_layout_passes=False),
        scratch_types=[
            pltpu.VMEM((8,), i32),
            pltpu.VMEM((NEP,), i32),
            pltpu.VMEM((NEP,), i32),
            pltpu.VMEM((XV,), i32),
            pltpu.VMEM((2, NEP), f32),
            pltpu.VMEM((2, NEP), f32),
            pltpu.VMEM((2, NEP), f32),
            pltpu.VMEM((2, NEP), f32),
            pltpu.VMEM((2, BW), f32),
            pltpu.VMEM((2, BW), f32),
            pltpu.VMEM((2, BW), f32),
            pltpu.VMEM((2, BW), f32),
            pltpu.VMEM((2 * XW + 24,), f32),
            pltpu.VMEM((X13,), f32),
            pltpu.VMEM((X13,), f32),
            pltpu.VMEM((X13,), f32),
            pltpu.VMEM((X13,), f32),
            pltpu.VMEM((208,), f32),
            pltpu.VMEM((208,), f32),
            pltpu.VMEM((208,), f32),
            pltpu.VMEM((XW + XV,), f32),
            pltpu.SemaphoreType.DMA,
            pltpu.SemaphoreType.DMA,
        ],
    )(_body)
    return fn(xp, ind, ei, ej, nt, wr, wd, wra, wda, br, bd, bra, bda)


def kernel(inputs, ind, edge_index, weight_react, weight_diff, weight_react_a,
           weight_diff_a, bias_reaction, bias_diffusion, bias_reaction_a,
           bias_diffusion_a):
    xp = inputs[:, 0, :, :].reshape(64 * XW)
    ep = jnp.pad(edge_index, ((0, 0), (0, NEP - N_EDGES)))
    nt = jnp.arange(NW * L, dtype=jnp.int32) // NF
    padw = lambda w: jnp.pad(w, ((0, 0), (0, NEP - N_EDGES)))
    padb = lambda b: jnp.pad(b, ((0, 0), (0, BW - N_NODES)))
    ind8 = jnp.pad(ind.astype(jnp.int32).reshape(32, 2), ((0, 0), (0, 6))).reshape(256)
    res = _run(xp, ind8, ep[0], ep[1], nt,
               padw(weight_react), padw(weight_diff),
               padw(weight_react_a), padw(weight_diff_a),
               padb(bias_reaction), padb(bias_diffusion),
               padb(bias_reaction_a), padb(bias_diffusion_a))
    return res.reshape(64, N_NODES, NF)
